# Initial kernel scaffold; baseline (speedup 1.0000x reference)
import jax
import jax.numpy as jnp
from jax.experimental import pallas as pl


def kernel(inputs, table):
    raise NotImplementedError("write your pallas kernel here")


# speedup vs baseline: 3.1235x; 3.1235x over previous
import functools

import jax
import jax.numpy as jnp
from jax import lax
from jax.experimental import pallas as pl
from jax.experimental.pallas import tpu as pltpu
from jax.experimental.pallas import tpu_sc as plsc

DIM = 128
CHUNK = 128
NUM_CORES = 2
NUM_SUBCORES = 16
NW = NUM_CORES * NUM_SUBCORES


def kernel(inputs, table):
    batch, seq = inputs.shape
    total = batch * seq
    rows_per_w = total // (NW * CHUNK)
    idx = inputs.reshape(NW, rows_per_w, CHUNK).astype(jnp.int32)

    mesh = plsc.VectorSubcoreMesh(core_axis_name="c", subcore_axis_name="s")

    @functools.partial(
        pl.kernel,
        mesh=mesh,
        out_type=jax.ShapeDtypeStruct((total, DIM), jnp.float32),
        scratch_types=[
            pltpu.VMEM((rows_per_w, CHUNK), jnp.int32),
            pltpu.VMEM((CHUNK, DIM), jnp.float32),
            pltpu.VMEM((CHUNK, DIM), jnp.float32),
            pltpu.SemaphoreType.DMA,
            pltpu.SemaphoreType.DMA,
        ],
    )
    def run(idx_hbm, table_hbm, out_hbm, idx_v, buf_a, buf_b, sem_a, sem_b):
        wid = lax.axis_index("s") * NUM_CORES + lax.axis_index("c")
        base = wid * rows_per_w * CHUNK
        pltpu.sync_copy(idx_hbm.at[wid], idx_v)

        pltpu.async_copy(table_hbm.at[idx_v.at[0]], buf_a, sem_a)

        n_pairs = rows_per_w // 2

        def body(i, carry):
            j0 = 2 * i
            j1 = j0 + 1
            pltpu.make_async_copy(table_hbm.at[idx_v.at[j0]], buf_a, sem_a).wait()
            pltpu.async_copy(table_hbm.at[idx_v.at[j1]], buf_b, sem_b)
            pltpu.sync_copy(buf_a, out_hbm.at[pl.ds(base + j0 * CHUNK, CHUNK)])
            pltpu.make_async_copy(table_hbm.at[idx_v.at[j1]], buf_b, sem_b).wait()

            @pl.when(i + 1 < n_pairs)
            def _():
                pltpu.async_copy(table_hbm.at[idx_v.at[j0 + 2]], buf_a, sem_a)

            pltpu.sync_copy(buf_b, out_hbm.at[pl.ds(base + j1 * CHUNK, CHUNK)])
            return carry

        lax.fori_loop(0, n_pairs, body, 0)

    out = run(idx, table)
    return out.reshape(batch, seq, DIM)


# speedup vs baseline: 3.2033x; 1.0255x over previous
import functools

import jax
import jax.numpy as jnp
from jax import lax
from jax.experimental import pallas as pl
from jax.experimental.pallas import tpu as pltpu
from jax.experimental.pallas import tpu_sc as plsc

DIM = 128
CHUNK = 128
NUM_CORES = 2
NUM_SUBCORES = 16
NW = NUM_CORES * NUM_SUBCORES


def kernel(inputs, table):
    batch, seq = inputs.shape
    total = batch * seq
    rows_per_w = total // (NW * CHUNK)
    idx = inputs.reshape(NW, rows_per_w, CHUNK).astype(jnp.int32)

    mesh = plsc.VectorSubcoreMesh(core_axis_name="c", subcore_axis_name="s")

    @functools.partial(
        pl.kernel,
        mesh=mesh,
        out_type=jax.ShapeDtypeStruct((total, DIM), jnp.float32),
        scratch_types=[
            pltpu.VMEM((rows_per_w, CHUNK), jnp.int32),
            pltpu.VMEM((CHUNK, DIM), jnp.float32),
            pltpu.VMEM((CHUNK, DIM), jnp.float32),
            pltpu.SemaphoreType.DMA,
            pltpu.SemaphoreType.DMA,
            pltpu.SemaphoreType.DMA,
            pltpu.SemaphoreType.DMA,
        ],
    )
    def run(idx_hbm, table_hbm, out_hbm, idx_v, buf_a, buf_b,
            gsem_a, gsem_b, osem_a, osem_b):
        wid = lax.axis_index("s") * NUM_CORES + lax.axis_index("c")
        base = wid * rows_per_w * CHUNK
        pltpu.sync_copy(idx_hbm.at[wid], idx_v)

        pltpu.async_copy(table_hbm.at[idx_v.at[0]], buf_a, gsem_a)
        pltpu.async_copy(table_hbm.at[idx_v.at[1]], buf_b, gsem_b)

        n_pairs = rows_per_w // 2

        def body(i, carry):
            j0 = 2 * i
            j1 = j0 + 1
            pltpu.make_async_copy(table_hbm.at[idx_v.at[j0]], buf_a, gsem_a).wait()
            oa = out_hbm.at[pl.ds(base + j0 * CHUNK, CHUNK)]
            pltpu.async_copy(buf_a, oa, osem_a)
            pltpu.make_async_copy(table_hbm.at[idx_v.at[j1]], buf_b, gsem_b).wait()
            ob = out_hbm.at[pl.ds(base + j1 * CHUNK, CHUNK)]
            pltpu.async_copy(buf_b, ob, osem_b)
            pltpu.make_async_copy(buf_a, oa, osem_a).wait()
            pltpu.make_async_copy(buf_b, ob, osem_b).wait()

            @pl.when(i + 1 < n_pairs)
            def _():
                pltpu.async_copy(table_hbm.at[idx_v.at[j0 + 2]], buf_a, gsem_a)
                pltpu.async_copy(table_hbm.at[idx_v.at[j1 + 2]], buf_b, gsem_b)

            return carry

        lax.fori_loop(0, n_pairs, body, 0)

    out = run(idx, table)
    return out.reshape(batch, seq, DIM)


# speedup vs baseline: 3.2595x; 1.0175x over previous
import functools

import jax
import jax.numpy as jnp
from jax import lax
from jax.experimental import pallas as pl
from jax.experimental.pallas import tpu as pltpu
from jax.experimental.pallas import tpu_sc as plsc

DIM = 128
CHUNK = 256
NUM_CORES = 2
NUM_SUBCORES = 16
NW = NUM_CORES * NUM_SUBCORES


def kernel(inputs, table):
    batch, seq = inputs.shape
    total = batch * seq
    per_w = total // NW
    n_chunks = per_w // CHUNK
    idx = inputs.reshape(total).astype(jnp.int32)

    mesh = plsc.VectorSubcoreMesh(core_axis_name="c", subcore_axis_name="s")

    @functools.partial(
        pl.kernel,
        mesh=mesh,
        out_type=jax.ShapeDtypeStruct((total, DIM), jnp.float32),
        scratch_types=[
            pltpu.VMEM((per_w,), jnp.int32),
            pltpu.VMEM((CHUNK, DIM), jnp.float32),
            pltpu.VMEM((CHUNK, DIM), jnp.float32),
            pltpu.SemaphoreType.DMA,
            pltpu.SemaphoreType.DMA,
            pltpu.SemaphoreType.DMA,
            pltpu.SemaphoreType.DMA,
        ],
    )
    def run(idx_hbm, table_hbm, out_hbm, idx_v, buf_a, buf_b,
            gsem_a, gsem_b, osem_a, osem_b):
        wid = lax.axis_index("s") * NUM_CORES + lax.axis_index("c")
        base = wid * per_w
        pltpu.sync_copy(idx_hbm.at[pl.ds(base, per_w)], idx_v)

        def g_src(j):
            return table_hbm.at[idx_v.at[pl.ds(j * CHUNK, CHUNK)]]

        def o_dst(j):
            return out_hbm.at[pl.ds(base + j * CHUNK, CHUNK)]

        pltpu.async_copy(g_src(0), buf_a, gsem_a)
        pltpu.async_copy(g_src(1), buf_b, gsem_b)

        def body(i, carry):
            j0 = 2 * i
            j1 = j0 + 1
            pltpu.make_async_copy(g_src(j0), buf_a, gsem_a).wait()
            pltpu.async_copy(buf_a, o_dst(j0), osem_a)
            pltpu.make_async_copy(g_src(j1), buf_b, gsem_b).wait()
            pltpu.async_copy(buf_b, o_dst(j1), osem_b)
            pltpu.make_async_copy(buf_a, o_dst(j0), osem_a).wait()
            pltpu.async_copy(g_src(j0 + 2), buf_a, gsem_a)
            pltpu.make_async_copy(buf_b, o_dst(j1), osem_b).wait()

            @pl.when(j1 + 2 < n_chunks)
            def _():
                pltpu.async_copy(g_src(j1 + 2), buf_b, gsem_b)

            return carry

        lax.fori_loop(0, (n_chunks - 1) // 2, body, 0)

        pltpu.make_async_copy(g_src(n_chunks - 1), buf_a, gsem_a).wait()
        pltpu.sync_copy(buf_a, o_dst(n_chunks - 1))

    out = run(idx, table)
    return out.reshape(batch, seq, DIM)


# speedup vs baseline: 5.7557x; 1.7659x over previous
import functools

import jax
import jax.numpy as jnp
from jax import lax
from jax.experimental import pallas as pl
from jax.experimental.pallas import tpu as pltpu
from jax.experimental.pallas import tpu_sc as plsc

DIM = 128
NB = 4
NUM_CORES = 2
NUM_SUBCORES = 16
NW = NUM_CORES * NUM_SUBCORES


def kernel(inputs, table):
    batch, seq = inputs.shape
    b_per_w = batch // NW
    n_groups = b_per_w // NB
    idx = inputs.astype(jnp.int32)

    mesh = plsc.VectorSubcoreMesh(core_axis_name="c", subcore_axis_name="s")

    @functools.partial(
        pl.kernel,
        mesh=mesh,
        out_type=jax.ShapeDtypeStruct((batch, seq, DIM), jnp.float32),
        scratch_types=[
            pltpu.VMEM((b_per_w, seq), jnp.int32),
            pltpu.VMEM((NB, seq, DIM), jnp.float32),
            pltpu.VMEM((NB, seq, DIM), jnp.float32),
            pltpu.SemaphoreType.DMA,
            pltpu.SemaphoreType.DMA,
            pltpu.SemaphoreType.DMA,
            pltpu.SemaphoreType.DMA,
        ],
    )
    def run(idx_hbm, table_hbm, out_hbm, idx_v, buf_a, buf_b,
            gsem_a, gsem_b, osem_a, osem_b):
        wid = lax.axis_index("s") * NUM_CORES + lax.axis_index("c")
        b0 = wid * b_per_w
        pltpu.sync_copy(idx_hbm.at[pl.ds(b0, b_per_w)], idx_v)

        def fire_gathers(g, buf, sem):
            for k in range(NB):
                pltpu.async_copy(
                    table_hbm.at[idx_v.at[g * NB + k]], buf.at[k], sem)

        def drain_gathers(g, buf, sem):
            for k in range(NB):
                pltpu.make_async_copy(
                    table_hbm.at[idx_v.at[g * NB + k]], buf.at[k], sem).wait()

        def o_dst(g):
            return out_hbm.at[pl.ds(b0 + g * NB, NB)]

        fire_gathers(0, buf_a, gsem_a)
        fire_gathers(1, buf_b, gsem_b)

        def body(i, carry):
            g0 = 2 * i
            g1 = g0 + 1
            drain_gathers(g0, buf_a, gsem_a)
            pltpu.async_copy(buf_a, o_dst(g0), osem_a)
            drain_gathers(g1, buf_b, gsem_b)
            pltpu.async_copy(buf_b, o_dst(g1), osem_b)
            pltpu.make_async_copy(buf_a, o_dst(g0), osem_a).wait()

            @pl.when(g0 + 2 < n_groups)
            def _():
                fire_gathers(g0 + 2, buf_a, gsem_a)

            pltpu.make_async_copy(buf_b, o_dst(g1), osem_b).wait()

            @pl.when(g1 + 2 < n_groups)
            def _():
                fire_gathers(g1 + 2, buf_b, gsem_b)

            return carry

        lax.fori_loop(0, n_groups // 2, body, 0)

    return run(idx, table)


# speedup vs baseline: 9.8924x; 1.7187x over previous
import functools

import jax
import jax.numpy as jnp
from jax import lax
from jax.experimental import pallas as pl
from jax.experimental.pallas import tpu as pltpu
from jax.experimental.pallas import tpu_sc as plsc

DIM = 128
CHUNK = 256
NUM_CORES = 2
NUM_SUBCORES = 16
NW = NUM_CORES * NUM_SUBCORES


def kernel(inputs, table):
    batch, seq = inputs.shape
    total = batch * seq
    per_w = total // NW
    n_chunks = per_w // CHUNK
    idx = inputs.T.reshape(total).astype(jnp.int32)

    mesh = plsc.VectorSubcoreMesh(core_axis_name="c", subcore_axis_name="s")

    @functools.partial(
        pl.kernel,
        mesh=mesh,
        out_type=jax.ShapeDtypeStruct((total, DIM), jnp.float32),
        scratch_types=[
            pltpu.VMEM((per_w,), jnp.int32),
            pltpu.VMEM((CHUNK, DIM), jnp.float32),
            pltpu.VMEM((CHUNK, DIM), jnp.float32),
            pltpu.SemaphoreType.DMA,
            pltpu.SemaphoreType.DMA,
            pltpu.SemaphoreType.DMA,
            pltpu.SemaphoreType.DMA,
        ],
    )
    def run(idx_hbm, table_hbm, out_hbm, idx_v, buf_a, buf_b,
            gsem_a, gsem_b, osem_a, osem_b):
        wid = lax.axis_index("s") * NUM_CORES + lax.axis_index("c")
        base = wid * per_w
        pltpu.sync_copy(idx_hbm.at[pl.ds(base, per_w)], idx_v)

        def g_src(j):
            return table_hbm.at[idx_v.at[pl.ds(j * CHUNK, CHUNK)]]

        def o_dst(j):
            return out_hbm.at[pl.ds(base + j * CHUNK, CHUNK)]

        pltpu.async_copy(g_src(0), buf_a, gsem_a)
        pltpu.async_copy(g_src(1), buf_b, gsem_b)

        def body(i, carry):
            j0 = 2 * i
            j1 = j0 + 1
            pltpu.make_async_copy(g_src(j0), buf_a, gsem_a).wait()
            pltpu.async_copy(buf_a, o_dst(j0), osem_a)
            pltpu.make_async_copy(g_src(j1), buf_b, gsem_b).wait()
            pltpu.async_copy(buf_b, o_dst(j1), osem_b)
            pltpu.make_async_copy(buf_a, o_dst(j0), osem_a).wait()
            pltpu.async_copy(g_src(j0 + 2), buf_a, gsem_a)
            pltpu.make_async_copy(buf_b, o_dst(j1), osem_b).wait()

            @pl.when(j1 + 2 < n_chunks)
            def _():
                pltpu.async_copy(g_src(j1 + 2), buf_b, gsem_b)

            return carry

        lax.fori_loop(0, (n_chunks - 1) // 2, body, 0)

        pltpu.make_async_copy(g_src(n_chunks - 1), buf_a, gsem_a).wait()
        pltpu.sync_copy(buf_a, o_dst(n_chunks - 1))

    out = run(idx, table)
    return out.reshape(seq, batch, DIM).transpose(1, 0, 2)
